# baseline (speedup 1.0000x reference)
import math

import jax
import jax.numpy as jnp
from jax.experimental import pallas as pl
from jax.experimental.pallas import tpu as pltpu

_T = 2048
_DM = 1024
_H, _DH = 16, 64
_HI, _DI = 4, 64
_TOPK = 64
_BQ = 256
_CB = 256
_NBLK = _T // _BQ

_NEG_INF = float("-inf")
_SENT_ENC = -1065353217


def _bdot(a, b, dims=(((1,), (0,)), ((), ()))):
    return jax.lax.dot_general(a.astype(jnp.bfloat16), b.astype(jnp.bfloat16),
                               dims, preferred_element_type=jnp.float32)


def _proj_kernel(x_ref, wq_ref, wk_ref, wv_ref, wqi_ref, wki_ref, ww_ref,
                 q_ref, k_ref, v_ref, qi_ref, ki_ref, wsm_ref):
    x = x_ref[...]
    bf = jnp.bfloat16
    q_ref[...] = _bdot(x, wq_ref[...]).astype(bf)
    k_ref[...] = _bdot(x, wk_ref[...]).astype(bf)
    v_ref[...] = _bdot(x, wv_ref[...]).astype(bf)
    qi_ref[...] = _bdot(x, wqi_ref[...]).astype(bf)
    ki_ref[...] = _bdot(x, wki_ref[...]).astype(bf)
    wl = _bdot(x, ww_ref[...])
    lane = jax.lax.broadcasted_iota(jnp.int32, wl.shape, 1)
    wl = jnp.where(lane < _HI, wl, _NEG_INF)
    m = jnp.max(wl, axis=-1, keepdims=True)
    e = jnp.exp(wl - m)
    wsm_ref[...] = (e / jnp.sum(e, axis=-1, keepdims=True)).astype(bf)


def _mid_int32(lo, hi):
    return (lo >> 1) + (hi >> 1) + (lo & hi & 1)


def _attn_kernel(q_ref, qi_ref, wsm_ref, k_ref, v_ref, ki_ref, wo_ref, bo_ref,
                 out_ref, enc_ref):
    i = pl.program_id(0)
    bq, t, cb = _BQ, _T, _CB
    nj = i + 1
    row = i * bq + jax.lax.broadcasted_iota(jnp.int32, (bq, cb), 0)
    wsm = wsm_ref[...].astype(jnp.float32)
    qi_hs = [qi_ref[:, h * _DI:(h + 1) * _DI] for h in range(_HI)]

    enc_ref[...] = jnp.full((bq, t), _SENT_ENC, jnp.int32)

    def score_chunk(j, _):
        off = pl.multiple_of(j * cb, cb)
        kij = ki_ref[pl.ds(off, cb), :_DI]
        s = jnp.zeros((bq, cb), jnp.float32)
        for h in range(_HI):
            dots = _bdot(qi_hs[h], kij, (((1,), (1,)), ((), ())))
            raw = jnp.maximum(dots * (1.0 / math.sqrt(_DI)), 0.0)
            raw = raw.astype(jnp.bfloat16).astype(jnp.float32)
            s = s + wsm[:, h][:, None] * raw
        colc = j * cb + jax.lax.broadcasted_iota(jnp.int32, (bq, cb), 1)
        s = jnp.where(colc <= row, s, -1.0)
        bits = jax.lax.bitcast_convert_type(s, jnp.int32)
        enc_ref[:, pl.ds(off, cb)] = jnp.where(bits < 0, bits ^ 0x7FFFFFFF,
                                               bits)
        return 0

    jax.lax.fori_loop(0, nj, score_chunk, 0)
    enc = enc_ref[...]
    col = jax.lax.broadcasted_iota(jnp.int32, (bq, t), 1)

    lo = jnp.full((bq, 1), _SENT_ENC - 1, jnp.int32)
    hi = jnp.full((bq, 1), 0x7F800000, jnp.int32)
    for _ in range(32):
        mid = _mid_int32(lo, hi)
        cnt = jnp.sum((enc > mid).astype(jnp.int32), axis=-1, keepdims=True)
        gt = cnt >= _TOPK
        lo = jnp.where(gt, mid, lo)
        hi = jnp.where(gt, hi, mid)
    kth = hi

    cnt_gt = jnp.sum((enc > kth).astype(jnp.int32), axis=-1, keepdims=True)
    need = _TOPK - cnt_gt
    is_tie = (enc == kth).astype(jnp.int32)
    clo = jnp.zeros((bq, 1), jnp.int32)
    chi = jnp.full((bq, 1), t, jnp.int32)
    for _ in range(12):
        mid = _mid_int32(clo, chi)
        cnt = jnp.sum(is_tie * (col < mid).astype(jnp.int32),
                      axis=-1, keepdims=True)
        ge = cnt >= need
        chi = jnp.where(ge, mid, chi)
        clo = jnp.where(ge, clo, mid)
    cutoff = chi

    scale = 1.0 / math.sqrt(_DH)
    ctx_parts = []
    for h in range(_H):
        q_h = q_ref[:, h * _DH:(h + 1) * _DH]

        def att_chunk(j, carry):
            m, l, acc = carry
            off = pl.multiple_of(j * cb, cb)
            kj = k_ref[pl.ds(off, cb), h * _DH:(h + 1) * _DH]
            vj = v_ref[pl.ds(off, cb), h * _DH:(h + 1) * _DH]
            logits = _bdot(q_h, kj, (((1,), (1,)), ((), ()))) * scale
            encc = enc_ref[:, pl.ds(off, cb)]
            colc = j * cb + jax.lax.broadcasted_iota(jnp.int32, (bq, cb), 1)
            selc = ((encc > kth) | ((encc == kth) & (colc < cutoff))) \
                & (colc <= row)
            logits = jnp.where(selc, logits, _NEG_INF)
            mc = jnp.max(logits, axis=-1, keepdims=True)
            mn = jnp.maximum(m, mc)
            alpha = jnp.where(m > _NEG_INF, jnp.exp(m - mn), 0.0)
            p = jnp.where(logits > _NEG_INF, jnp.exp(logits - mn), 0.0)
            l = l * alpha + jnp.sum(p, axis=-1, keepdims=True)
            acc = acc * alpha + _bdot(p, vj)
            return mn, l, acc

        m0 = jnp.full((bq, 1), _NEG_INF, jnp.float32)
        l0 = jnp.zeros((bq, 1), jnp.float32)
        a0 = jnp.zeros((bq, _DH), jnp.float32)
        _, l, acc = jax.lax.fori_loop(0, nj, att_chunk, (m0, l0, a0))
        ctx_parts.append(acc / l)
    ctx = jnp.concatenate(ctx_parts, axis=-1)
    out_ref[...] = _bdot(ctx, wo_ref[...]) + bo_ref[...]


@jax.jit
def _run(x2d, Wq, Wk, Wv, Wo, bo, Wqi, Wki_p, Ww_p):
    f32, bf = jnp.float32, jnp.bfloat16
    row_blk = lambda w: pl.BlockSpec((_BQ, w), lambda i: (i, 0))
    full = lambda a, b: pl.BlockSpec((a, b), lambda i: (0, 0))

    q, k, v, qi, ki, wsm = pl.pallas_call(
        _proj_kernel,
        grid=(_NBLK,),
        in_specs=[row_blk(_DM), full(_DM, _DM), full(_DM, _DM), full(_DM, _DM),
                  full(_DM, _HI * _DI), full(_DM, 128), full(_DM, 128)],
        out_specs=[row_blk(_DM), row_blk(_DM), row_blk(_DM),
                   row_blk(_HI * _DI), row_blk(128), row_blk(128)],
        out_shape=[jax.ShapeDtypeStruct((_T, _DM), bf),
                   jax.ShapeDtypeStruct((_T, _DM), bf),
                   jax.ShapeDtypeStruct((_T, _DM), bf),
                   jax.ShapeDtypeStruct((_T, _HI * _DI), bf),
                   jax.ShapeDtypeStruct((_T, 128), bf),
                   jax.ShapeDtypeStruct((_T, 128), bf)],
    )(x2d, Wq, Wk, Wv, Wqi, Wki_p, Ww_p)

    out = pl.pallas_call(
        _attn_kernel,
        grid=(_NBLK,),
        in_specs=[row_blk(_DM), row_blk(_HI * _DI), row_blk(128),
                  full(_T, _DM), full(_T, _DM), full(_T, 128),
                  full(_DM, _DM), pl.BlockSpec((1, _DM), lambda i: (0, 0))],
        out_specs=row_blk(_DM),
        out_shape=jax.ShapeDtypeStruct((_T, _DM), f32),
        scratch_shapes=[pltpu.VMEM((_BQ, _T), jnp.int32)],
    )(q, qi, wsm, k, v, ki, Wo, bo.reshape(1, _DM))
    return out


def kernel(x, Wq, Wk, Wv, Wo, bo, Wqi, Wki, Ww):
    b, t, _ = x.shape
    x2d = x.reshape(t, _DM)
    Wki_p = jnp.pad(Wki, ((0, 0), (0, 128 - Wki.shape[1])))
    Ww_p = jnp.pad(Ww, ((0, 0), (0, 128 - Ww.shape[1])))
    out = _run(x2d, Wq, Wk, Wv, Wo, bo, Wqi, Wki_p, Ww_p)
    return out.reshape(b, t, _DM)


# speedup vs baseline: 1.3930x; 1.3930x over previous
import math

import jax
import jax.numpy as jnp
from jax.experimental import pallas as pl

_T = 2048
_DM = 1024
_H, _DH = 16, 64
_HI, _DI = 4, 64
_TOPK = 64
_BQ = 256
_NBLK = _T // _BQ

_NEG_INF = float("-inf")
_SENT_ENC = -1065353217


def _bdot(a, b, dims=(((1,), (0,)), ((), ()))):
    return jax.lax.dot_general(a.astype(jnp.bfloat16), b.astype(jnp.bfloat16),
                               dims, preferred_element_type=jnp.float32)


def _proj_kernel(x_ref, wq_ref, wk_ref, wv_ref, wqi_ref, wki_ref, ww_ref,
                 q_ref, k_ref, v_ref, qi_ref, ki_ref, wsm_ref):
    x = x_ref[...]
    bf = jnp.bfloat16
    q_ref[...] = _bdot(x, wq_ref[...]).astype(bf)
    k_ref[...] = _bdot(x, wk_ref[...]).astype(bf)
    v_ref[...] = _bdot(x, wv_ref[...]).astype(bf)
    qi_ref[...] = _bdot(x, wqi_ref[...]).astype(bf)
    ki_ref[...] = _bdot(x, wki_ref[...]).astype(bf)
    wl = _bdot(x, ww_ref[...])
    lane = jax.lax.broadcasted_iota(jnp.int32, wl.shape, 1)
    wl = jnp.where(lane < _HI, wl, _NEG_INF)
    m = jnp.max(wl, axis=-1, keepdims=True)
    e = jnp.exp(wl - m)
    wsm_ref[...] = (e / jnp.sum(e, axis=-1, keepdims=True)).astype(bf)


def _mid_int32(lo, hi):
    return (lo >> 1) + (hi >> 1) + (lo & hi & 1)


def _attn_kernel(q_ref, qi_ref, wsm_ref, k_ref, v_ref, ki_ref, wo_ref, bo_ref,
                 out_ref):
    i = pl.program_id(0)
    bq, t = _BQ, _T

    ki = ki_ref[:, :_DI]
    wsm = wsm_ref[...].astype(jnp.float32)
    s_idx = jnp.zeros((bq, t), jnp.float32)
    for h in range(_HI):
        qi_h = qi_ref[:, h * _DI:(h + 1) * _DI]
        dots = _bdot(qi_h, ki, (((1,), (1,)), ((), ())))
        raw = jnp.maximum(dots * (1.0 / math.sqrt(_DI)), 0.0)
        raw = raw.astype(jnp.bfloat16).astype(jnp.float32)
        s_idx = s_idx + wsm[:, h][:, None] * raw

    col = jax.lax.broadcasted_iota(jnp.int32, (bq, t), 1)
    row = i * bq + jax.lax.broadcasted_iota(jnp.int32, (bq, t), 0)
    causal = col <= row
    s_m = jnp.where(causal, s_idx, -1.0)

    s_bits = jax.lax.bitcast_convert_type(s_m, jnp.int32)
    enc = jnp.where(s_bits < 0, s_bits ^ 0x7FFFFFFF, s_bits)

    lo = jnp.full((bq, 1), _SENT_ENC - 1, jnp.int32)
    hi = jnp.full((bq, 1), 0x7F800000, jnp.int32)
    for _ in range(32):
        mid = _mid_int32(lo, hi)
        cnt = jnp.sum((enc > mid).astype(jnp.int32), axis=-1, keepdims=True)
        gt = cnt >= _TOPK
        lo = jnp.where(gt, mid, lo)
        hi = jnp.where(gt, hi, mid)
    kth = hi

    cnt_gt = jnp.sum((enc > kth).astype(jnp.int32), axis=-1, keepdims=True)
    need = _TOPK - cnt_gt
    is_tie = (enc == kth).astype(jnp.int32)
    clo = jnp.zeros((bq, 1), jnp.int32)
    chi = jnp.full((bq, 1), t, jnp.int32)
    for _ in range(12):
        mid = _mid_int32(clo, chi)
        cnt = jnp.sum(is_tie * (col < mid).astype(jnp.int32),
                      axis=-1, keepdims=True)
        ge = cnt >= need
        chi = jnp.where(ge, mid, chi)
        clo = jnp.where(ge, clo, mid)
    cutoff = chi

    sel = ((enc > kth) | ((enc == kth) & (col < cutoff))) & causal

    scale = 1.0 / math.sqrt(_DH)
    ctx_parts = []
    for h in range(_H):
        q_h = q_ref[:, h * _DH:(h + 1) * _DH]
        k_h = k_ref[:, h * _DH:(h + 1) * _DH]
        logits = _bdot(q_h, k_h, (((1,), (1,)), ((), ())))
        logits = jnp.where(sel, logits * scale, _NEG_INF)
        m = jnp.max(logits, axis=-1, keepdims=True)
        p = jnp.exp(logits - m)
        p = p / jnp.sum(p, axis=-1, keepdims=True)
        ctx_parts.append(_bdot(p, v_ref[:, h * _DH:(h + 1) * _DH]))
    ctx = jnp.concatenate(ctx_parts, axis=-1)
    out_ref[...] = _bdot(ctx, wo_ref[...]) + bo_ref[...]


@jax.jit
def _run(x2d, Wq, Wk, Wv, Wo, bo, Wqi, Wki_p, Ww_p):
    f32, bf = jnp.float32, jnp.bfloat16
    row_blk = lambda w: pl.BlockSpec((_BQ, w), lambda i: (i, 0))
    full = lambda a, b: pl.BlockSpec((a, b), lambda i: (0, 0))

    q, k, v, qi, ki, wsm = pl.pallas_call(
        _proj_kernel,
        grid=(_NBLK,),
        in_specs=[row_blk(_DM), full(_DM, _DM), full(_DM, _DM), full(_DM, _DM),
                  full(_DM, _HI * _DI), full(_DM, 128), full(_DM, 128)],
        out_specs=[row_blk(_DM), row_blk(_DM), row_blk(_DM),
                   row_blk(_HI * _DI), row_blk(128), row_blk(128)],
        out_shape=[jax.ShapeDtypeStruct((_T, _DM), bf),
                   jax.ShapeDtypeStruct((_T, _DM), bf),
                   jax.ShapeDtypeStruct((_T, _DM), bf),
                   jax.ShapeDtypeStruct((_T, _HI * _DI), bf),
                   jax.ShapeDtypeStruct((_T, 128), bf),
                   jax.ShapeDtypeStruct((_T, 128), bf)],
    )(x2d, Wq, Wk, Wv, Wqi, Wki_p, Ww_p)

    out = pl.pallas_call(
        _attn_kernel,
        grid=(_NBLK,),
        in_specs=[row_blk(_DM), row_blk(_HI * _DI), row_blk(128),
                  full(_T, _DM), full(_T, _DM), full(_T, 128),
                  full(_DM, _DM), pl.BlockSpec((1, _DM), lambda i: (0, 0))],
        out_specs=row_blk(_DM),
        out_shape=jax.ShapeDtypeStruct((_T, _DM), f32),
    )(q, qi, wsm, k, v, ki, Wo, bo.reshape(1, _DM))
    return out


def kernel(x, Wq, Wk, Wv, Wo, bo, Wqi, Wki, Ww):
    b, t, _ = x.shape
    x2d = x.reshape(t, _DM)
    Wki_p = jnp.pad(Wki, ((0, 0), (0, 128 - Wki.shape[1])))
    Ww_p = jnp.pad(Ww, ((0, 0), (0, 128 - Ww.shape[1])))
    out = _run(x2d, Wq, Wk, Wv, Wo, bo, Wqi, Wki_p, Ww_p)
    return out.reshape(b, t, _DM)


# speedup vs baseline: 1.5433x; 1.1079x over previous
import functools
import math

import jax
import jax.numpy as jnp
from jax.experimental import pallas as pl

_T = 2048
_DM = 1024
_H, _DH = 16, 64
_HI, _DI = 4, 64
_TOPK = 64
_BQ = 256
_NBLK = _T // _BQ

_NEG_INF = float("-inf")
_SENT_ENC = -1065353217


def _bdot(a, b, dims=(((1,), (0,)), ((), ()))):
    return jax.lax.dot_general(a.astype(jnp.bfloat16), b.astype(jnp.bfloat16),
                               dims, preferred_element_type=jnp.float32)


def _proj_kernel(x_ref, wq_ref, wk_ref, wv_ref, wqi_ref, wki_ref, ww_ref,
                 q_ref, k_ref, v_ref, qi_ref, ki_ref, wsm_ref):
    x = x_ref[...]
    bf = jnp.bfloat16
    q_ref[...] = _bdot(x, wq_ref[...]).astype(bf)
    k_ref[...] = _bdot(x, wk_ref[...]).astype(bf)
    v_ref[...] = _bdot(x, wv_ref[...]).astype(bf)
    qi_ref[...] = _bdot(x, wqi_ref[...]).astype(bf)
    ki_ref[...] = _bdot(x, wki_ref[...]).astype(bf)
    wl = _bdot(x, ww_ref[...])
    lane = jax.lax.broadcasted_iota(jnp.int32, wl.shape, 1)
    wl = jnp.where(lane < _HI, wl, _NEG_INF)
    m = jnp.max(wl, axis=-1, keepdims=True)
    e = jnp.exp(wl - m)
    wsm_ref[...] = (e / jnp.sum(e, axis=-1, keepdims=True)).astype(bf)


def _mid_int32(lo, hi):
    return (lo >> 1) + (hi >> 1) + (lo & hi & 1)


def _attn_kernel(base_blk, width, q_ref, qi_ref, wsm_ref, k_ref, v_ref,
                 ki_ref, wo_ref, bo_ref, out_ref):
    i = pl.program_id(0)
    bq, t = _BQ, width

    ki = ki_ref[:, :_DI]
    wsm = wsm_ref[...].astype(jnp.float32)
    s_idx = jnp.zeros((bq, t), jnp.float32)
    for h in range(_HI):
        qi_h = qi_ref[:, h * _DI:(h + 1) * _DI]
        dots = _bdot(qi_h, ki, (((1,), (1,)), ((), ())))
        raw = jnp.maximum(dots * (1.0 / math.sqrt(_DI)), 0.0)
        raw = raw.astype(jnp.bfloat16).astype(jnp.float32)
        s_idx = s_idx + wsm[:, h][:, None] * raw

    col = jax.lax.broadcasted_iota(jnp.int32, (bq, t), 1)
    row = (base_blk + i) * bq + jax.lax.broadcasted_iota(jnp.int32, (bq, t), 0)
    causal = col <= row
    s_m = jnp.where(causal, s_idx, -1.0)

    s_bits = jax.lax.bitcast_convert_type(s_m, jnp.int32)
    enc = jnp.where(s_bits < 0, s_bits ^ 0x7FFFFFFF, s_bits)

    lo = jnp.full((bq, 1), _SENT_ENC - 1, jnp.int32)
    hi = jnp.full((bq, 1), 0x7F800000, jnp.int32)
    for _ in range(32):
        mid = _mid_int32(lo, hi)
        cnt = jnp.sum((enc > mid).astype(jnp.int32), axis=-1, keepdims=True)
        gt = cnt >= _TOPK
        lo = jnp.where(gt, mid, lo)
        hi = jnp.where(gt, hi, mid)
    kth = hi

    cnt_gt = jnp.sum((enc > kth).astype(jnp.int32), axis=-1, keepdims=True)
    need = _TOPK - cnt_gt
    is_tie = (enc == kth).astype(jnp.int32)
    clo = jnp.zeros((bq, 1), jnp.int32)
    chi = jnp.full((bq, 1), t, jnp.int32)
    nbis = max(1, (t).bit_length())
    for _ in range(nbis):
        mid = _mid_int32(clo, chi)
        cnt = jnp.sum(is_tie * (col < mid).astype(jnp.int32),
                      axis=-1, keepdims=True)
        ge = cnt >= need
        chi = jnp.where(ge, mid, chi)
        clo = jnp.where(ge, clo, mid)
    cutoff = chi

    sel = ((enc > kth) | ((enc == kth) & (col < cutoff))) & causal

    scale = 1.0 / math.sqrt(_DH)
    ctx_parts = []
    for h in range(_H):
        q_h = q_ref[:, h * _DH:(h + 1) * _DH]
        k_h = k_ref[:, h * _DH:(h + 1) * _DH]
        logits = _bdot(q_h, k_h, (((1,), (1,)), ((), ())))
        logits = jnp.where(sel, logits * scale, _NEG_INF)
        m = jnp.max(logits, axis=-1, keepdims=True)
        p = jnp.exp(logits - m)
        p = p / jnp.sum(p, axis=-1, keepdims=True)
        ctx_parts.append(_bdot(p, v_ref[:, h * _DH:(h + 1) * _DH]))
    ctx = jnp.concatenate(ctx_parts, axis=-1)
    out_ref[...] = _bdot(ctx, wo_ref[...]) + bo_ref[...]


@jax.jit
def _run(x2d, Wq, Wk, Wv, Wo, bo, Wqi, Wki_p, Ww_p):
    f32, bf = jnp.float32, jnp.bfloat16
    row_blk = lambda w: pl.BlockSpec((_BQ, w), lambda i: (i, 0))
    full = lambda a, b: pl.BlockSpec((a, b), lambda i: (0, 0))

    q, k, v, qi, ki, wsm = pl.pallas_call(
        _proj_kernel,
        grid=(_NBLK,),
        in_specs=[row_blk(_DM), full(_DM, _DM), full(_DM, _DM), full(_DM, _DM),
                  full(_DM, _HI * _DI), full(_DM, 128), full(_DM, 128)],
        out_specs=[row_blk(_DM), row_blk(_DM), row_blk(_DM),
                   row_blk(_HI * _DI), row_blk(128), row_blk(128)],
        out_shape=[jax.ShapeDtypeStruct((_T, _DM), bf),
                   jax.ShapeDtypeStruct((_T, _DM), bf),
                   jax.ShapeDtypeStruct((_T, _DM), bf),
                   jax.ShapeDtypeStruct((_T, _HI * _DI), bf),
                   jax.ShapeDtypeStruct((_T, 128), bf),
                   jax.ShapeDtypeStruct((_T, 128), bf)],
    )(x2d, Wq, Wk, Wv, Wqi, Wki_p, Ww_p)

    bo2 = bo.reshape(1, _DM)
    outs = []
    blks_per_call = 2
    for c in range(4):
        base = c * blks_per_call
        width = (base + blks_per_call) * _BQ
        rb = lambda w, b=base: pl.BlockSpec((_BQ, w), lambda i, b=b: (b + i, 0))
        out_c = pl.pallas_call(
            functools.partial(_attn_kernel, base, width),
            grid=(blks_per_call,),
            in_specs=[rb(_DM), rb(_HI * _DI), rb(128),
                      pl.BlockSpec((width, _DM), lambda i: (0, 0)),
                      pl.BlockSpec((width, _DM), lambda i: (0, 0)),
                      pl.BlockSpec((width, 128), lambda i: (0, 0)),
                      full(_DM, _DM), pl.BlockSpec((1, _DM), lambda i: (0, 0))],
            out_specs=pl.BlockSpec((_BQ, _DM), lambda i: (i, 0)),
            out_shape=jax.ShapeDtypeStruct((blks_per_call * _BQ, _DM), f32),
        )(q, qi, wsm, k, v, ki, Wo, bo2)
        outs.append(out_c)
    return jnp.concatenate(outs, axis=0)


def kernel(x, Wq, Wk, Wv, Wo, bo, Wqi, Wki, Ww):
    b, t, _ = x.shape
    x2d = x.reshape(t, _DM)
    Wki_p = jnp.pad(Wki, ((0, 0), (0, 128 - Wki.shape[1])))
    Ww_p = jnp.pad(Ww, ((0, 0), (0, 128 - Ww.shape[1])))
    out = _run(x2d, Wq, Wk, Wv, Wo, bo, Wqi, Wki_p, Ww_p)
    return out.reshape(b, t, _DM)
